# baseline (speedup 1.0000x reference)
import functools
import jax
import jax.numpy as jnp
from jax import lax
from jax.experimental import pallas as pl
from jax.experimental.pallas import tpu as pltpu

from jax.experimental.pallas import tpu_sc as plsc

B_, T_, D_ = 2, 2048, 1024
H_, K_, S_ = 8, 8, 512
HD_ = D_ // H_
N_ = B_ * T_

_NEG = float('-inf')


def _topk_iter(s, iota, k):
    L = s.shape[1]
    vals, idxs = [], []
    for _ in range(k):
        m = jnp.max(s, axis=1, keepdims=True)
        eq = s == m
        iv = jnp.min(jnp.where(eq, iota, L), axis=1, keepdims=True)
        vals.append(m)
        idxs.append(iv)
        s = jnp.where(iota == iv, _NEG, s)
    return jnp.concatenate(vals, axis=1), jnp.concatenate(idxs, axis=1)


def _router_body(x_ref, w_ref, ka_ref, kb_ref, h_ref, wout_ref, iout_ref):
    BT = x_ref.shape[0]
    h = jnp.dot(x_ref[...], w_ref[...].T, preferred_element_type=jnp.float32)
    h_ref[...] = h
    iota_s = lax.broadcasted_iota(jnp.int32, (1, S_), 1)
    iota_p = lax.broadcasted_iota(jnp.int32, (1, K_ * K_), 1)
    w_parts, i_parts = [], []
    for hd in range(H_):
        hh = h[:, hd * HD_:(hd + 1) * HD_]
        sa = lax.dot_general(hh, ka_ref[hd], (((1,), (1,)), ((), ())),
                             preferred_element_type=jnp.float32)
        sb = lax.dot_general(hh, kb_ref[hd], (((1,), (1,)), ((), ())),
                             preferred_element_type=jnp.float32)
        va, ia = _topk_iter(sa, iota_s, K_)
        vb, ib = _topk_iter(sb, iota_s, K_)
        pv = (va[:, :, None] + vb[:, None, :]).reshape(BT, K_ * K_)
        pi = (ia[:, :, None] * S_ + ib[:, None, :]).reshape(BT, K_ * K_)
        tv, tpos = _topk_iter(pv, iota_p, K_)
        ti = []
        for j in range(K_):
            sel = iota_p == tpos[:, j:j + 1]
            ti.append(jnp.min(jnp.where(sel, pi, jnp.int32(2147483647)),
                              axis=1, keepdims=True))
        ti = jnp.concatenate(ti, axis=1)
        mx = jnp.max(tv, axis=1, keepdims=True)
        e = jnp.exp(tv - mx)
        w = e / jnp.sum(e, axis=1, keepdims=True)
        w_parts.append(w)
        i_parts.append(ti)
    wout_ref[...] = jnp.concatenate(w_parts, axis=1)
    iout_ref[...] = jnp.concatenate(i_parts, axis=1)


def _router(x2d, W_in, keys_a, keys_b):
    BT = 256
    return pl.pallas_call(
        _router_body,
        grid=(N_ // BT,),
        in_specs=[
            pl.BlockSpec((BT, D_), lambda i: (i, 0)),
            pl.BlockSpec((D_, D_), lambda i: (0, 0)),
            pl.BlockSpec((H_, S_, HD_), lambda i: (0, 0, 0)),
            pl.BlockSpec((H_, S_, HD_), lambda i: (0, 0, 0)),
        ],
        out_specs=[
            pl.BlockSpec((BT, D_), lambda i: (i, 0)),
            pl.BlockSpec((BT, H_ * K_), lambda i: (i, 0)),
            pl.BlockSpec((BT, H_ * K_), lambda i: (i, 0)),
        ],
        out_shape=[
            jax.ShapeDtypeStruct((N_, D_), jnp.float32),
            jax.ShapeDtypeStruct((N_, H_ * K_), jnp.float32),
            jax.ShapeDtypeStruct((N_, H_ * K_), jnp.int32),
        ],
    )(x2d, W_in, keys_a, keys_b)


NH_ = N_ * H_
NW_ = 32
THW_ = NH_ // NW_
CC_ = 16
CK_ = CC_ * K_
NCH_ = THW_ // CC_
_NLANE = 8


_GDN = lax.GatherDimensionNumbers(offset_dims=(), collapsed_slice_dims=(0,),
                                  start_index_map=(0,))


def _perm(vec, idx):
    return lax.gather(vec, idx[:, None], _GDN, slice_sizes=(1,),
                      mode=lax.GatherScatterMode.PROMISE_IN_BOUNDS)


def _lane_bcast(vec, lane):
    return _perm(vec, jnp.full((16,), lane, jnp.int32))


def _allreduce16(vec, lane_iota):
    for off in (8, 4, 2, 1):
        vec = vec + _perm(vec, lane_iota ^ off)
    return vec


def _expert_body(h_hbm, idx_hbm, w_hbm, u_hbm, v_hbm, out_hbm,
                 idx_v, w_v, h_v, u_rows, v_rows, out_v, sem_u, sem_v):
    wid = jax.lax.axis_index("s") * 2 + jax.lax.axis_index("c")
    base_th0 = wid * THW_
    lane_iota = jax.lax.broadcasted_iota(jnp.int32, (16,), 0)

    def chunk(ci, carry):
        base_th = base_th0 + ci * CC_
        base_ck = base_th * K_
        pltpu.sync_copy(idx_hbm.at[pl.ds(base_ck, CK_)], idx_v)
        pltpu.sync_copy(w_hbm.at[pl.ds(base_ck, CK_)], w_v)
        pltpu.sync_copy(h_hbm.at[pl.ds(base_th, CC_)], h_v)
        cp_u = pltpu.async_copy(u_hbm.at[idx_v], u_rows, sem_u)
        cp_v = pltpu.async_copy(v_hbm.at[idx_v], v_rows, sem_v)
        cp_u.wait()
        cp_v.wait()
        for cp in range(CC_ // 2):
            w16 = w_v[pl.ds(cp * 16, 16)]
            for sub in range(2):
                c = cp * 2 + sub
                hr = [h_v[c, pl.ds(16 * j, 16)] for j in range(_NLANE)]
                acc = [jnp.zeros((16,), jnp.float32) for _ in range(_NLANE)]
                for k in range(K_):
                    row = c * K_ + k
                    p = hr[0] * u_rows[row, pl.ds(0, 16)]
                    for j in range(1, _NLANE):
                        p = p + hr[j] * u_rows[row, pl.ds(16 * j, 16)]
                    dot = _allreduce16(p, lane_iota)
                    act = 1.0 / (1.0 + jnp.exp(-dot))
                    alpha = act * _lane_bcast(w16, sub * 8 + k)
                    for j in range(_NLANE):
                        acc[j] = acc[j] + alpha * v_rows[row, pl.ds(16 * j, 16)]
                for j in range(_NLANE):
                    out_v[c, pl.ds(16 * j, 16)] = acc[j]
        pltpu.sync_copy(out_v, out_hbm.at[pl.ds(base_th, CC_)])
        return carry

    jax.lax.fori_loop(0, NCH_, chunk, 0)


def _expert_sc(hflat, idx_flat, w_flat, expert_u, expert_v):
    mesh = plsc.VectorSubcoreMesh(core_axis_name="c", subcore_axis_name="s")
    f = functools.partial(
        pl.kernel,
        mesh=mesh,
        out_type=jax.ShapeDtypeStruct((NH_, HD_), jnp.float32),
        scratch_types=[
            pltpu.VMEM((CK_,), jnp.int32),
            pltpu.VMEM((CK_,), jnp.float32),
            pltpu.VMEM((CC_, HD_), jnp.float32),
            pltpu.VMEM((CK_, HD_), jnp.float32),
            pltpu.VMEM((CK_, HD_), jnp.float32),
            pltpu.VMEM((CC_, HD_), jnp.float32),
            pltpu.SemaphoreType.DMA,
            pltpu.SemaphoreType.DMA,
        ],
    )(_expert_body)
    return f(hflat, idx_flat, w_flat, expert_u, expert_v)


def _out_ln_body(m_ref, w_ref, g_ref, b_ref, o_ref):
    y = jnp.dot(m_ref[...], w_ref[...].T, preferred_element_type=jnp.float32)
    mu = jnp.mean(y, axis=-1, keepdims=True)
    var = jnp.mean((y - mu) ** 2, axis=-1, keepdims=True)
    yn = (y - mu) * lax.rsqrt(var + 1e-5)
    o_ref[...] = yn * g_ref[...] + b_ref[...]


def _out_ln(merged, W_out, gamma, beta):
    BT = 512
    return pl.pallas_call(
        _out_ln_body,
        grid=(N_ // BT,),
        in_specs=[
            pl.BlockSpec((BT, D_), lambda i: (i, 0)),
            pl.BlockSpec((D_, D_), lambda i: (0, 0)),
            pl.BlockSpec((1, D_), lambda i: (0, 0)),
            pl.BlockSpec((1, D_), lambda i: (0, 0)),
        ],
        out_specs=pl.BlockSpec((BT, D_), lambda i: (i, 0)),
        out_shape=jax.ShapeDtypeStruct((N_, D_), jnp.float32),
    )(merged, W_out, gamma.reshape(1, D_), beta.reshape(1, D_))


def kernel(x, W_in, keys_a, keys_b, expert_u, expert_v, W_out, gamma, beta):
    h2d, w_flat, i_flat = _router(x.reshape(N_, D_), W_in, keys_a, keys_b)
    expert_out = _expert_sc(h2d.reshape(NH_, HD_), i_flat.reshape(NH_ * K_),
                            w_flat.reshape(NH_ * K_), expert_u, expert_v)
    merged = expert_out.reshape(N_, D_)
    out = _out_ln(merged, W_out, gamma, beta)
    return out.reshape(B_, T_, D_)


# speedup vs baseline: 1.2636x; 1.2636x over previous
import functools
import jax
import jax.numpy as jnp
from jax import lax
from jax.experimental import pallas as pl
from jax.experimental.pallas import tpu as pltpu

from jax.experimental.pallas import tpu_sc as plsc

B_, T_, D_ = 2, 2048, 1024
H_, K_, S_ = 8, 8, 512
HD_ = D_ // H_
N_ = B_ * T_

_NEG = float('-inf')


def _topk_iter(s, iota, k):
    L = float(s.shape[1])
    vals, idxs = [], []
    for _ in range(k):
        m = jnp.max(s, axis=1, keepdims=True)
        iv = jnp.min(jnp.where(s == m, iota, L), axis=1, keepdims=True)
        vals.append(m)
        idxs.append(iv)
        s = jnp.where(iota == iv, _NEG, s)
    return jnp.concatenate(vals, axis=1), jnp.concatenate(idxs, axis=1)


def _router_body(x_ref, w_ref, ka_ref, kb_ref, h_ref, wout_ref, iout_ref):
    BT = x_ref.shape[0]
    h = jnp.dot(x_ref[...], w_ref[...].T, preferred_element_type=jnp.float32)
    h_ref[...] = h
    iota_s = lax.broadcasted_iota(jnp.int32, (1, S_), 1).astype(jnp.float32)
    iota_p = lax.broadcasted_iota(jnp.int32, (1, K_ * K_), 1).astype(jnp.float32)
    w_parts, i_parts = [], []
    for hd in range(H_):
        hh = h[:, hd * HD_:(hd + 1) * HD_]
        sa = lax.dot_general(hh, ka_ref[hd], (((1,), (1,)), ((), ())),
                             preferred_element_type=jnp.float32)
        sb = lax.dot_general(hh, kb_ref[hd], (((1,), (1,)), ((), ())),
                             preferred_element_type=jnp.float32)
        va, ia = _topk_iter(sa, iota_s, K_)
        vb, ib = _topk_iter(sb, iota_s, K_)
        pv = (va[:, :, None] + vb[:, None, :]).reshape(BT, K_ * K_)
        pi = (ia[:, :, None] * S_ + ib[:, None, :]).reshape(BT, K_ * K_)
        tv, tpos = _topk_iter(pv, iota_p, K_)
        ti = []
        for j in range(K_):
            sel = iota_p == tpos[:, j:j + 1]
            ti.append(jnp.min(jnp.where(sel, pi, jnp.float32(16777216.0)),
                              axis=1, keepdims=True))
        ti = jnp.concatenate(ti, axis=1)
        mx = jnp.max(tv, axis=1, keepdims=True)
        e = jnp.exp(tv - mx)
        w = e / jnp.sum(e, axis=1, keepdims=True)
        w_parts.append(w)
        i_parts.append(ti)
    wout_ref[...] = jnp.concatenate(w_parts, axis=1)
    iout_ref[...] = jnp.concatenate(i_parts, axis=1).astype(jnp.int32)


def _router(x2d, W_in, keys_a, keys_b):
    BT = 256
    return pl.pallas_call(
        _router_body,
        grid=(N_ // BT,),
        in_specs=[
            pl.BlockSpec((BT, D_), lambda i: (i, 0)),
            pl.BlockSpec((D_, D_), lambda i: (0, 0)),
            pl.BlockSpec((H_, S_, HD_), lambda i: (0, 0, 0)),
            pl.BlockSpec((H_, S_, HD_), lambda i: (0, 0, 0)),
        ],
        out_specs=[
            pl.BlockSpec((BT, D_), lambda i: (i, 0)),
            pl.BlockSpec((BT, H_ * K_), lambda i: (i, 0)),
            pl.BlockSpec((BT, H_ * K_), lambda i: (i, 0)),
        ],
        out_shape=[
            jax.ShapeDtypeStruct((N_, D_), jnp.float32),
            jax.ShapeDtypeStruct((N_, H_ * K_), jnp.float32),
            jax.ShapeDtypeStruct((N_, H_ * K_), jnp.int32),
        ],
    )(x2d, W_in, keys_a, keys_b)


NH_ = N_ * H_
NW_ = 32
THW_ = NH_ // NW_
CC_ = 16
CK_ = CC_ * K_
NCH_ = THW_ // CC_
_NLANE = 8


_GDN = lax.GatherDimensionNumbers(offset_dims=(), collapsed_slice_dims=(0,),
                                  start_index_map=(0,))


def _perm(vec, idx):
    return lax.gather(vec, idx[:, None], _GDN, slice_sizes=(1,),
                      mode=lax.GatherScatterMode.PROMISE_IN_BOUNDS)


def _lane_bcast(vec, lane):
    return _perm(vec, jnp.full((16,), lane, jnp.int32))


def _allreduce16(vec, lane_iota):
    for off in (8, 4, 2, 1):
        vec = vec + _perm(vec, lane_iota ^ off)
    return vec


def _expert_body(h_hbm, idx_hbm, w_hbm, u_hbm, v_hbm, out_hbm,
                 idx_v, w_v, h_v, u_rows, v_rows, out_v, sem_u, sem_v):
    wid = jax.lax.axis_index("s") * 2 + jax.lax.axis_index("c")
    base_th0 = wid * THW_
    lane_iota = jax.lax.broadcasted_iota(jnp.int32, (16,), 0)

    def chunk(ci, carry):
        base_th = base_th0 + ci * CC_
        base_ck = base_th * K_
        pltpu.sync_copy(idx_hbm.at[pl.ds(base_ck, CK_)], idx_v)
        pltpu.sync_copy(w_hbm.at[pl.ds(base_ck, CK_)], w_v)
        pltpu.sync_copy(h_hbm.at[pl.ds(base_th, CC_)], h_v)
        cp_u = pltpu.async_copy(u_hbm.at[idx_v], u_rows, sem_u)
        cp_v = pltpu.async_copy(v_hbm.at[idx_v], v_rows, sem_v)
        cp_u.wait()
        cp_v.wait()
        for cp in range(CC_ // 2):
            w16 = w_v[pl.ds(cp * 16, 16)]
            for sub in range(2):
                c = cp * 2 + sub
                hr = [h_v[c, pl.ds(16 * j, 16)] for j in range(_NLANE)]
                acc = [jnp.zeros((16,), jnp.float32) for _ in range(_NLANE)]
                for k in range(K_):
                    row = c * K_ + k
                    p = hr[0] * u_rows[row, pl.ds(0, 16)]
                    for j in range(1, _NLANE):
                        p = p + hr[j] * u_rows[row, pl.ds(16 * j, 16)]
                    dot = _allreduce16(p, lane_iota)
                    act = 1.0 / (1.0 + jnp.exp(-dot))
                    alpha = act * _lane_bcast(w16, sub * 8 + k)
                    for j in range(_NLANE):
                        acc[j] = acc[j] + alpha * v_rows[row, pl.ds(16 * j, 16)]
                for j in range(_NLANE):
                    out_v[c, pl.ds(16 * j, 16)] = acc[j]
        pltpu.sync_copy(out_v, out_hbm.at[pl.ds(base_th, CC_)])
        return carry

    jax.lax.fori_loop(0, NCH_, chunk, 0)


def _expert_sc(hflat, idx_flat, w_flat, expert_u, expert_v):
    mesh = plsc.VectorSubcoreMesh(core_axis_name="c", subcore_axis_name="s")
    f = functools.partial(
        pl.kernel,
        mesh=mesh,
        out_type=jax.ShapeDtypeStruct((NH_, HD_), jnp.float32),
        scratch_types=[
            pltpu.VMEM((CK_,), jnp.int32),
            pltpu.VMEM((CK_,), jnp.float32),
            pltpu.VMEM((CC_, HD_), jnp.float32),
            pltpu.VMEM((CK_, HD_), jnp.float32),
            pltpu.VMEM((CK_, HD_), jnp.float32),
            pltpu.VMEM((CC_, HD_), jnp.float32),
            pltpu.SemaphoreType.DMA,
            pltpu.SemaphoreType.DMA,
        ],
    )(_expert_body)
    return f(hflat, idx_flat, w_flat, expert_u, expert_v)


def _out_ln_body(m_ref, w_ref, g_ref, b_ref, o_ref):
    y = jnp.dot(m_ref[...], w_ref[...].T, preferred_element_type=jnp.float32)
    mu = jnp.mean(y, axis=-1, keepdims=True)
    var = jnp.mean((y - mu) ** 2, axis=-1, keepdims=True)
    yn = (y - mu) * lax.rsqrt(var + 1e-5)
    o_ref[...] = yn * g_ref[...] + b_ref[...]


def _out_ln(merged, W_out, gamma, beta):
    BT = 256
    return pl.pallas_call(
        _out_ln_body,
        grid=(N_ // BT,),
        in_specs=[
            pl.BlockSpec((BT, D_), lambda i: (i, 0)),
            pl.BlockSpec((D_, D_), lambda i: (0, 0)),
            pl.BlockSpec((1, D_), lambda i: (0, 0)),
            pl.BlockSpec((1, D_), lambda i: (0, 0)),
        ],
        out_specs=pl.BlockSpec((BT, D_), lambda i: (i, 0)),
        out_shape=jax.ShapeDtypeStruct((N_, D_), jnp.float32),
    )(merged, W_out, gamma.reshape(1, D_), beta.reshape(1, D_))


def kernel(x, W_in, keys_a, keys_b, expert_u, expert_v, W_out, gamma, beta):
    h2d, w_flat, i_flat = _router(x.reshape(N_, D_), W_in, keys_a, keys_b)
    expert_out = _expert_sc(h2d.reshape(NH_, HD_), i_flat.reshape(NH_ * K_),
                            w_flat.reshape(NH_ * K_), expert_u, expert_v)
    merged = expert_out.reshape(N_, D_)
    out = _out_ln(merged, W_out, gamma, beta)
    return out.reshape(B_, T_, D_)


# speedup vs baseline: 1.5595x; 1.2342x over previous
import functools
import jax
import jax.numpy as jnp
from jax import lax
from jax.experimental import pallas as pl
from jax.experimental.pallas import tpu as pltpu

from jax.experimental.pallas import tpu_sc as plsc

B_, T_, D_ = 2, 2048, 1024
H_, K_, S_ = 8, 8, 512
HD_ = D_ // H_
N_ = B_ * T_

_NEG = float('-inf')


def _topk_iter(s, iota, k):
    L = float(s.shape[1])
    vals, idxs = [], []
    for _ in range(k):
        m = jnp.max(s, axis=1, keepdims=True)
        iv = jnp.min(jnp.where(s == m, iota, L), axis=1, keepdims=True)
        vals.append(m)
        idxs.append(iv)
        s = jnp.where(iota == iv, _NEG, s)
    return jnp.concatenate(vals, axis=1), jnp.concatenate(idxs, axis=1)


def _router_body(x_ref, w_ref, ka_ref, kb_ref, h_ref, wout_ref, iout_ref):
    BT = x_ref.shape[0]
    h = jnp.dot(x_ref[...], w_ref[...].T, preferred_element_type=jnp.float32)
    h_ref[...] = h
    iota_s = lax.broadcasted_iota(jnp.int32, (1, S_), 1).astype(jnp.float32)
    iota_p = lax.broadcasted_iota(jnp.int32, (1, K_ * K_), 1).astype(jnp.float32)
    w_parts, i_parts = [], []
    for hd in range(H_):
        hh = h[:, hd * HD_:(hd + 1) * HD_]
        sa = lax.dot_general(hh, ka_ref[hd], (((1,), (1,)), ((), ())),
                             preferred_element_type=jnp.float32)
        sb = lax.dot_general(hh, kb_ref[hd], (((1,), (1,)), ((), ())),
                             preferred_element_type=jnp.float32)
        va, ia = _topk_iter(sa, iota_s, K_)
        vb, ib = _topk_iter(sb, iota_s, K_)
        pv = (va[:, :, None] + vb[:, None, :]).reshape(BT, K_ * K_)
        pi = (ia[:, :, None] * S_ + ib[:, None, :]).reshape(BT, K_ * K_)
        tv, tpos = _topk_iter(pv, iota_p, K_)
        ti = []
        for j in range(K_):
            sel = iota_p == tpos[:, j:j + 1]
            ti.append(jnp.min(jnp.where(sel, pi, jnp.float32(16777216.0)),
                              axis=1, keepdims=True))
        ti = jnp.concatenate(ti, axis=1)
        mx = jnp.max(tv, axis=1, keepdims=True)
        e = jnp.exp(tv - mx)
        w = e / jnp.sum(e, axis=1, keepdims=True)
        w_parts.append(jnp.concatenate([w[:, q:q + 1] for q in _WPERM], axis=1))
        i_parts.append(ti)
    wout_ref[...] = jnp.concatenate(w_parts, axis=1)
    iout_ref[...] = jnp.concatenate(i_parts, axis=1).astype(jnp.int32)


def _router(x2d, W_in, keys_a, keys_b):
    BT = 256
    return pl.pallas_call(
        _router_body,
        grid=(N_ // BT,),
        in_specs=[
            pl.BlockSpec((BT, D_), lambda i: (i, 0)),
            pl.BlockSpec((D_, D_), lambda i: (0, 0)),
            pl.BlockSpec((H_, S_, HD_), lambda i: (0, 0, 0)),
            pl.BlockSpec((H_, S_, HD_), lambda i: (0, 0, 0)),
        ],
        out_specs=[
            pl.BlockSpec((BT, D_), lambda i: (i, 0)),
            pl.BlockSpec((BT, H_ * 16), lambda i: (i, 0)),
            pl.BlockSpec((BT, H_ * K_), lambda i: (i, 0)),
        ],
        out_shape=[
            jax.ShapeDtypeStruct((N_, D_), jnp.float32),
            jax.ShapeDtypeStruct((N_, H_ * 16), jnp.float32),
            jax.ShapeDtypeStruct((N_, H_ * K_), jnp.int32),
        ],
    )(x2d, W_in, keys_a, keys_b)


NH_ = N_ * H_
NW_ = 32
THW_ = NH_ // NW_
CC_ = 16
CK_ = CC_ * K_
NCH_ = THW_ // CC_
_NLANE = 8


_GDN = lax.GatherDimensionNumbers(offset_dims=(), collapsed_slice_dims=(0,),
                                  start_index_map=(0,))


def _perm(vec, idx):
    return lax.gather(vec, idx[:, None], _GDN, slice_sizes=(1,),
                      mode=lax.GatherScatterMode.PROMISE_IN_BOUNDS)


def _lane_bcast(vec, lane):
    return _perm(vec, jnp.full((16,), lane, jnp.int32))


def _allreduce16(vec, lane_iota):
    for off in (8, 4, 2, 1):
        vec = vec + _perm(vec, lane_iota ^ off)
    return vec


_LOK = [0, 8, 4, 12, 2, 10, 6, 14]
_WPERM = [0, 0, 4, 4, 2, 2, 6, 6, 1, 1, 5, 5, 3, 3, 7, 7]


def _expert_body(h_hbm, idx_hbm, w_hbm, u_hbm, v_hbm, out_hbm, *scr):
    idx_b = scr[0:4]
    w_b = scr[4:8]
    h_b = scr[8:12]
    u_b = scr[12:14]
    v_b = scr[14:16]
    o_b = scr[16:18]
    si = scr[18:22]
    sg = scr[22:24]
    so = scr[24:26]

    wid = jax.lax.axis_index("s") * 2 + jax.lax.axis_index("c")
    base0 = wid * THW_
    lane = jax.lax.broadcasted_iota(jnp.int32, (16,), 0)
    m8 = (lane & 8) == 0
    m4 = (lane & 4) == 0
    m2 = (lane & 2) == 0
    px8 = lane ^ 8
    px4 = lane ^ 4
    px2 = lane ^ 2
    px1 = lane ^ 1

    def in_copy(ci, b):
        bt = base0 + ci * CC_
        pltpu.async_copy(idx_hbm.at[pl.ds(bt * K_, CK_)], idx_b[b], si[b])
        pltpu.async_copy(w_hbm.at[pl.ds(bt * 16, CC_ * 16)], w_b[b], si[b])
        pltpu.async_copy(h_hbm.at[pl.ds(bt, CC_)], h_b[b], si[b])

    def in_wait(b):
        pltpu.make_async_copy(idx_hbm.at[pl.ds(0, CK_)], idx_b[b], si[b]).wait()
        pltpu.make_async_copy(w_hbm.at[pl.ds(0, CC_ * 16)], w_b[b], si[b]).wait()
        pltpu.make_async_copy(h_hbm.at[pl.ds(0, CC_)], h_b[b], si[b]).wait()

    def gather_start(b, g):
        pltpu.async_copy(u_hbm.at[idx_b[b]], u_b[g], sg[g])
        pltpu.async_copy(v_hbm.at[idx_b[b]], v_b[g], sg[g])

    def gather_wait(g):
        pltpu.make_async_copy(u_hbm.at[pl.ds(0, CK_)], u_b[g], sg[g]).wait()
        pltpu.make_async_copy(v_hbm.at[pl.ds(0, CK_)], v_b[g], sg[g]).wait()

    def out_start(ci, g):
        bt = base0 + ci * CC_
        pltpu.async_copy(o_b[g], out_hbm.at[pl.ds(bt, CC_)], so[g])

    def out_wait(g):
        pltpu.make_async_copy(out_hbm.at[pl.ds(0, CC_)], o_b[g], so[g]).wait()

    def compute(hh, wv, uu, vv, oo):
        def comp_c(c, carry):
            hr = [hh[c, pl.ds(16 * j, 16)] for j in range(_NLANE)]
            wv16 = wv[pl.ds(c * 16, 16)]
            a = []
            for k in range(K_):
                row = c * K_ + k
                p = hr[0] * uu[row, pl.ds(0, 16)]
                for j in range(1, _NLANE):
                    p = p + hr[j] * uu[row, pl.ds(16 * j, 16)]
                a.append(p + _perm(p, px8))
            b0 = jnp.where(m8, a[0], a[1])
            b1 = jnp.where(m8, a[2], a[3])
            b2 = jnp.where(m8, a[4], a[5])
            b3 = jnp.where(m8, a[6], a[7])
            c0 = b0 + _perm(b0, px4)
            c1 = b1 + _perm(b1, px4)
            c2 = b2 + _perm(b2, px4)
            c3 = b3 + _perm(b3, px4)
            d0 = jnp.where(m4, c0, c1)
            d1 = jnp.where(m4, c2, c3)
            e0 = d0 + _perm(d0, px2)
            e1 = d1 + _perm(d1, px2)
            f = jnp.where(m2, e0, e1)
            dots = f + _perm(f, px1)
            alpha = wv16 / (1.0 + jnp.exp(-dots))
            al = [_lane_bcast(alpha, _LOK[k]) for k in range(K_)]
            for j in range(_NLANE):
                s = al[0] * vv[c * K_, pl.ds(16 * j, 16)]
                for k in range(1, K_):
                    s = s + al[k] * vv[c * K_ + k, pl.ds(16 * j, 16)]
                oo[c, pl.ds(16 * j, 16)] = s
            return carry
        jax.lax.fori_loop(0, CC_, comp_c, 0)

    in_copy(0, 0)
    in_copy(1, 1)
    in_copy(2, 2)
    in_wait(0)
    gather_start(0, 0)

    def outer(oi, carry):
        for ph in range(4):
            ci = oi * 4 + ph
            g = ph % 2
            gn = (ph + 1) % 2
            bn = (ph + 1) % 4

            @pl.when(ci + 1 < NCH_)
            def _():
                in_wait(bn)
                gather_start(bn, gn)

            @pl.when(ci + 3 < NCH_)
            def _():
                in_copy(ci + 3, (ph + 3) % 4)

            @pl.when(ci >= 2)
            def _():
                out_wait(g)

            gather_wait(g)
            compute(h_b[ph], w_b[ph], u_b[g], v_b[g], o_b[g])
            out_start(ci, g)
        return carry

    jax.lax.fori_loop(0, NCH_ // 4, outer, 0)
    out_wait(0)
    out_wait(1)


def _expert_sc(hflat, idx_flat, w_flat, expert_u, expert_v):
    mesh = plsc.VectorSubcoreMesh(core_axis_name="c", subcore_axis_name="s")
    scr = []
    scr += [pltpu.VMEM((CK_,), jnp.int32) for _ in range(4)]
    scr += [pltpu.VMEM((CC_ * 16,), jnp.float32) for _ in range(4)]
    scr += [pltpu.VMEM((CC_, HD_), jnp.float32) for _ in range(4)]
    scr += [pltpu.VMEM((CK_, HD_), jnp.float32) for _ in range(2)]
    scr += [pltpu.VMEM((CK_, HD_), jnp.float32) for _ in range(2)]
    scr += [pltpu.VMEM((CC_, HD_), jnp.float32) for _ in range(2)]
    scr += [pltpu.SemaphoreType.DMA for _ in range(8)]
    f = functools.partial(
        pl.kernel,
        mesh=mesh,
        out_type=jax.ShapeDtypeStruct((NH_, HD_), jnp.float32),
        scratch_types=scr,
    )(_expert_body)
    return f(hflat, idx_flat, w_flat, expert_u, expert_v)


def _out_ln_body(m_ref, w_ref, g_ref, b_ref, o_ref):
    y = jnp.dot(m_ref[...], w_ref[...].T, preferred_element_type=jnp.float32)
    mu = jnp.mean(y, axis=-1, keepdims=True)
    var = jnp.mean((y - mu) ** 2, axis=-1, keepdims=True)
    yn = (y - mu) * lax.rsqrt(var + 1e-5)
    o_ref[...] = yn * g_ref[...] + b_ref[...]


def _out_ln(merged, W_out, gamma, beta):
    BT = 256
    return pl.pallas_call(
        _out_ln_body,
        grid=(N_ // BT,),
        in_specs=[
            pl.BlockSpec((BT, D_), lambda i: (i, 0)),
            pl.BlockSpec((D_, D_), lambda i: (0, 0)),
            pl.BlockSpec((1, D_), lambda i: (0, 0)),
            pl.BlockSpec((1, D_), lambda i: (0, 0)),
        ],
        out_specs=pl.BlockSpec((BT, D_), lambda i: (i, 0)),
        out_shape=jax.ShapeDtypeStruct((N_, D_), jnp.float32),
    )(merged, W_out, gamma.reshape(1, D_), beta.reshape(1, D_))


def kernel(x, W_in, keys_a, keys_b, expert_u, expert_v, W_out, gamma, beta):
    h2d, w_flat, i_flat = _router(x.reshape(N_, D_), W_in, keys_a, keys_b)
    expert_out = _expert_sc(h2d.reshape(NH_, HD_), i_flat.reshape(NH_ * K_),
                            w_flat.reshape(NH_ * 16), expert_u, expert_v)
    merged = expert_out.reshape(N_, D_)
    out = _out_ln(merged, W_out, gamma, beta)
    return out.reshape(B_, T_, D_)


# speedup vs baseline: 2.3033x; 1.4770x over previous
import functools
import jax
import jax.numpy as jnp
from jax import lax
from jax.experimental import pallas as pl
from jax.experimental.pallas import tpu as pltpu

from jax.experimental.pallas import tpu_sc as plsc

B_, T_, D_ = 2, 2048, 1024
H_, K_, S_ = 8, 8, 512
HD_ = D_ // H_
N_ = B_ * T_

_NEG = float('-inf')


def _topk_iter(s, iota, k):
    L = float(s.shape[1])
    vals, idxs = [], []
    for _ in range(k):
        m = jnp.max(s, axis=1, keepdims=True)
        iv = jnp.min(jnp.where(s == m, iota, L), axis=1, keepdims=True)
        vals.append(m)
        idxs.append(iv)
        s = jnp.where(iota == iv, _NEG, s)
    return jnp.concatenate(vals, axis=1), jnp.concatenate(idxs, axis=1)


def _topk_pair_spread(sa, sb, iota, masks_a, masks_b):
    BT = sa.shape[0]
    z = jnp.zeros((BT, K_ * K_), jnp.float32)
    va, ia, vb, ib = z, z, z, z
    for i in range(K_):
        ma = jnp.max(sa, axis=1, keepdims=True)
        mb = jnp.max(sb, axis=1, keepdims=True)
        iva = jnp.min(jnp.where(sa == ma, iota, float(S_)), axis=1,
                      keepdims=True)
        ivb = jnp.min(jnp.where(sb == mb, iota, float(S_)), axis=1,
                      keepdims=True)
        va = jnp.where(masks_a[i], ma, va)
        ia = jnp.where(masks_a[i], iva, ia)
        vb = jnp.where(masks_b[i], mb, vb)
        ib = jnp.where(masks_b[i], ivb, ib)
        sa = jnp.where(iota == iva, _NEG, sa)
        sb = jnp.where(iota == ivb, _NEG, sb)
    return va, ia, vb, ib


def _router_body(x_ref, w_ref, ka_ref, kb_ref, h_ref, wout_ref, iout_ref):
    BT = x_ref.shape[0]
    h = jnp.dot(x_ref[...], w_ref[...].T, preferred_element_type=jnp.float32)
    h_ref[...] = h
    iota_s = lax.broadcasted_iota(jnp.int32, (1, S_), 1).astype(jnp.float32)
    iota_p = lax.broadcasted_iota(jnp.int32, (1, K_ * K_), 1).astype(jnp.float32)
    iota_p_i = lax.broadcasted_iota(jnp.int32, (1, K_ * K_), 1)
    masks_a = [(iota_p_i // K_) == i for i in range(K_)]
    masks_b = [(iota_p_i % K_) == i for i in range(K_)]
    w_parts, i_parts = [], []
    for hd in range(H_):
        hh = h[:, hd * HD_:(hd + 1) * HD_]
        sa = lax.dot_general(hh, ka_ref[hd], (((1,), (1,)), ((), ())),
                             preferred_element_type=jnp.float32)
        sb = lax.dot_general(hh, kb_ref[hd], (((1,), (1,)), ((), ())),
                             preferred_element_type=jnp.float32)
        va, ia, vb, ib = _topk_pair_spread(sa, sb, iota_s, masks_a, masks_b)
        pv = va + vb
        pi = ia * float(S_) + ib
        tv, tpos = _topk_iter(pv, iota_p, K_)
        ti = []
        for j in range(K_):
            sel = iota_p == tpos[:, j:j + 1]
            ti.append(jnp.min(jnp.where(sel, pi, jnp.float32(16777216.0)),
                              axis=1, keepdims=True))
        ti = jnp.concatenate(ti, axis=1)
        mx = jnp.max(tv, axis=1, keepdims=True)
        e = jnp.exp(tv - mx)
        w = e / jnp.sum(e, axis=1, keepdims=True)
        w_parts.append(jnp.concatenate([w[:, q:q + 1] for q in _WPERM], axis=1))
        i_parts.append(ti)
    wout_ref[...] = jnp.concatenate(w_parts, axis=1)
    iout_ref[...] = jnp.concatenate(i_parts, axis=1).astype(jnp.int32)


def _router(x2d, W_in, keys_a, keys_b):
    BT = 256
    return pl.pallas_call(
        _router_body,
        grid=(N_ // BT,),
        in_specs=[
            pl.BlockSpec((BT, D_), lambda i: (i, 0)),
            pl.BlockSpec((D_, D_), lambda i: (0, 0)),
            pl.BlockSpec((H_, S_, HD_), lambda i: (0, 0, 0)),
            pl.BlockSpec((H_, S_, HD_), lambda i: (0, 0, 0)),
        ],
        out_specs=[
            pl.BlockSpec((BT, D_), lambda i: (i, 0)),
            pl.BlockSpec((BT, H_ * 16), lambda i: (i, 0)),
            pl.BlockSpec((BT, H_ * K_), lambda i: (i, 0)),
        ],
        out_shape=[
            jax.ShapeDtypeStruct((N_, D_), jnp.float32),
            jax.ShapeDtypeStruct((N_, H_ * 16), jnp.float32),
            jax.ShapeDtypeStruct((N_, H_ * K_), jnp.int32),
        ],
    )(x2d, W_in, keys_a, keys_b)


NH_ = N_ * H_
NW_ = 32
THW_ = NH_ // NW_
CC_ = 16
CK_ = CC_ * K_
NCH_ = THW_ // CC_
_NLANE = 8


_GDN = lax.GatherDimensionNumbers(offset_dims=(), collapsed_slice_dims=(0,),
                                  start_index_map=(0,))


def _perm(vec, idx):
    return lax.gather(vec, idx[:, None], _GDN, slice_sizes=(1,),
                      mode=lax.GatherScatterMode.PROMISE_IN_BOUNDS)


def _lane_bcast(vec, lane):
    return _perm(vec, jnp.full((16,), lane, jnp.int32))


def _allreduce16(vec, lane_iota):
    for off in (8, 4, 2, 1):
        vec = vec + _perm(vec, lane_iota ^ off)
    return vec


_LOK = [0, 8, 4, 12, 2, 10, 6, 14]
_WPERM = [0, 0, 4, 4, 2, 2, 6, 6, 1, 1, 5, 5, 3, 3, 7, 7]


def _expert_body(h_hbm, idx_hbm, w_hbm, u_hbm, v_hbm, out_hbm, *scr):
    idx_b = scr[0:4]
    w_b = scr[4:8]
    h_b = scr[8:12]
    u_b = scr[12:14]
    v_b = scr[14:16]
    o_b = scr[16:18]
    si = scr[18:22]
    sg = scr[22:24]
    so = scr[24:26]

    wid = jax.lax.axis_index("s") * 2 + jax.lax.axis_index("c")
    base0 = wid * THW_
    lane = jax.lax.broadcasted_iota(jnp.int32, (16,), 0)
    m8 = (lane & 8) == 0
    m4 = (lane & 4) == 0
    m2 = (lane & 2) == 0
    px8 = lane ^ 8
    px4 = lane ^ 4
    px2 = lane ^ 2
    px1 = lane ^ 1

    def in_copy(ci, b):
        bt = base0 + ci * CC_
        pltpu.async_copy(idx_hbm.at[pl.ds(bt * K_, CK_)], idx_b[b], si[b])
        pltpu.async_copy(w_hbm.at[pl.ds(bt * 16, CC_ * 16)], w_b[b], si[b])
        pltpu.async_copy(h_hbm.at[pl.ds(bt, CC_)], h_b[b], si[b])

    def in_wait(b):
        pltpu.make_async_copy(idx_hbm.at[pl.ds(0, CK_)], idx_b[b], si[b]).wait()
        pltpu.make_async_copy(w_hbm.at[pl.ds(0, CC_ * 16)], w_b[b], si[b]).wait()
        pltpu.make_async_copy(h_hbm.at[pl.ds(0, CC_)], h_b[b], si[b]).wait()

    def gather_start(b, g):
        pltpu.async_copy(u_hbm.at[idx_b[b]], u_b[g], sg[g])
        pltpu.async_copy(v_hbm.at[idx_b[b]], v_b[g], sg[g])

    def gather_wait(g):
        pltpu.make_async_copy(u_hbm.at[pl.ds(0, CK_)], u_b[g], sg[g]).wait()
        pltpu.make_async_copy(v_hbm.at[pl.ds(0, CK_)], v_b[g], sg[g]).wait()

    def out_start(ci, g):
        bt = base0 + ci * CC_
        pltpu.async_copy(o_b[g], out_hbm.at[pl.ds(bt, CC_)], so[g])

    def out_wait(g):
        pltpu.make_async_copy(out_hbm.at[pl.ds(0, CC_)], o_b[g], so[g]).wait()

    def compute(hh, wv, uu, vv, oo):
        def comp_c(c, carry):
            hr = [hh[c, pl.ds(16 * j, 16)] for j in range(_NLANE)]
            wv16 = wv[pl.ds(c * 16, 16)]
            a = []
            for k in range(K_):
                row = c * K_ + k
                p = hr[0] * uu[row, pl.ds(0, 16)]
                for j in range(1, _NLANE):
                    p = p + hr[j] * uu[row, pl.ds(16 * j, 16)]
                a.append(p + _perm(p, px8))
            b0 = jnp.where(m8, a[0], a[1])
            b1 = jnp.where(m8, a[2], a[3])
            b2 = jnp.where(m8, a[4], a[5])
            b3 = jnp.where(m8, a[6], a[7])
            c0 = b0 + _perm(b0, px4)
            c1 = b1 + _perm(b1, px4)
            c2 = b2 + _perm(b2, px4)
            c3 = b3 + _perm(b3, px4)
            d0 = jnp.where(m4, c0, c1)
            d1 = jnp.where(m4, c2, c3)
            e0 = d0 + _perm(d0, px2)
            e1 = d1 + _perm(d1, px2)
            f = jnp.where(m2, e0, e1)
            dots = f + _perm(f, px1)
            alpha = wv16 / (1.0 + jnp.exp(-dots))
            al = [_lane_bcast(alpha, _LOK[k]) for k in range(K_)]
            for j in range(_NLANE):
                s = al[0] * vv[c * K_, pl.ds(16 * j, 16)]
                for k in range(1, K_):
                    s = s + al[k] * vv[c * K_ + k, pl.ds(16 * j, 16)]
                oo[c, pl.ds(16 * j, 16)] = s
            return carry
        jax.lax.fori_loop(0, CC_, comp_c, 0)

    in_copy(0, 0)
    in_copy(1, 1)
    in_copy(2, 2)
    in_wait(0)
    gather_start(0, 0)

    def outer(oi, carry):
        for ph in range(4):
            ci = oi * 4 + ph
            g = ph % 2
            gn = (ph + 1) % 2
            bn = (ph + 1) % 4

            @pl.when(ci + 1 < NCH_)
            def _():
                in_wait(bn)
                gather_start(bn, gn)

            @pl.when(ci + 3 < NCH_)
            def _():
                in_copy(ci + 3, (ph + 3) % 4)

            @pl.when(ci >= 2)
            def _():
                out_wait(g)

            gather_wait(g)
            compute(h_b[ph], w_b[ph], u_b[g], v_b[g], o_b[g])
            out_start(ci, g)
        return carry

    jax.lax.fori_loop(0, NCH_ // 4, outer, 0)
    out_wait(0)
    out_wait(1)


def _expert_sc(hflat, idx_flat, w_flat, expert_u, expert_v):
    mesh = plsc.VectorSubcoreMesh(core_axis_name="c", subcore_axis_name="s")
    scr = []
    scr += [pltpu.VMEM((CK_,), jnp.int32) for _ in range(4)]
    scr += [pltpu.VMEM((CC_ * 16,), jnp.float32) for _ in range(4)]
    scr += [pltpu.VMEM((CC_, HD_), jnp.float32) for _ in range(4)]
    scr += [pltpu.VMEM((CK_, HD_), jnp.float32) for _ in range(2)]
    scr += [pltpu.VMEM((CK_, HD_), jnp.float32) for _ in range(2)]
    scr += [pltpu.VMEM((CC_, HD_), jnp.float32) for _ in range(2)]
    scr += [pltpu.SemaphoreType.DMA for _ in range(8)]
    f = functools.partial(
        pl.kernel,
        mesh=mesh,
        out_type=jax.ShapeDtypeStruct((NH_, HD_), jnp.float32),
        scratch_types=scr,
    )(_expert_body)
    return f(hflat, idx_flat, w_flat, expert_u, expert_v)


def _out_ln_body(m_ref, w_ref, g_ref, b_ref, o_ref):
    y = jnp.dot(m_ref[...], w_ref[...].T, preferred_element_type=jnp.float32)
    mu = jnp.mean(y, axis=-1, keepdims=True)
    var = jnp.mean((y - mu) ** 2, axis=-1, keepdims=True)
    yn = (y - mu) * lax.rsqrt(var + 1e-5)
    o_ref[...] = yn * g_ref[...] + b_ref[...]


def _out_ln(merged, W_out, gamma, beta):
    BT = 256
    return pl.pallas_call(
        _out_ln_body,
        grid=(N_ // BT,),
        in_specs=[
            pl.BlockSpec((BT, D_), lambda i: (i, 0)),
            pl.BlockSpec((D_, D_), lambda i: (0, 0)),
            pl.BlockSpec((1, D_), lambda i: (0, 0)),
            pl.BlockSpec((1, D_), lambda i: (0, 0)),
        ],
        out_specs=pl.BlockSpec((BT, D_), lambda i: (i, 0)),
        out_shape=jax.ShapeDtypeStruct((N_, D_), jnp.float32),
    )(merged, W_out, gamma.reshape(1, D_), beta.reshape(1, D_))


def kernel(x, W_in, keys_a, keys_b, expert_u, expert_v, W_out, gamma, beta):
    h2d, w_flat, i_flat = _router(x.reshape(N_, D_), W_in, keys_a, keys_b)
    expert_out = _expert_sc(h2d.reshape(NH_, HD_), i_flat.reshape(NH_ * K_),
                            w_flat.reshape(NH_ * 16), expert_u, expert_v)
    merged = expert_out.reshape(N_, D_)
    out = _out_ln(merged, W_out, gamma, beta)
    return out.reshape(B_, T_, D_)
